# baseline (speedup 1.0000x reference)
import functools

import jax
import jax.numpy as jnp
from jax import lax
from jax.experimental import pallas as pl
from jax.experimental.pallas import tpu as pltpu
from jax.experimental.pallas import tpu_sc as plsc

M = 100000
D = 64
B = 16384

NC = 2
NS = 16
PASSES_PER_CORE = 2
MP = M // (NC * PASSES_PER_CORE)
NPAD = 512
BPT = B // NS
CHUNK = 128

ROWS_A = 1568
ROWS_B = MP - 15 * ROWS_A


def _body(mem_hbm, idx_hbm, val_hbm, out_hbm, slab, valbuf, idxbuf, locbuf):
    c = lax.axis_index("c")
    s = lax.axis_index("s")
    jbase = s * BPT

    pltpu.sync_copy(idx_hbm.at[pl.ds(jbase, BPT)], idxbuf)

    lane = lax.broadcasted_iota(jnp.int32, (16,), 0)

    for p in range(PASSES_PER_CORE):
        lo = (c * PASSES_PER_CORE + p) * MP

        off = lo + s * ROWS_A

        @pl.when(s < NS - 1)
        def _():
            pltpu.sync_copy(mem_hbm.at[pl.ds(off, ROWS_A)],
                            slab.at[pl.ds(s * ROWS_A, ROWS_A)])

        @pl.when(s == NS - 1)
        def _():
            pltpu.sync_copy(mem_hbm.at[pl.ds(off, ROWS_B)],
                            slab.at[pl.ds(s * ROWS_A, ROWS_B)])

        plsc.subcore_barrier()

        for v in range(BPT // 16):
            iv = idxbuf[pl.ds(v * 16, 16)]
            in_range = (iv >= lo) & (iv < lo + MP)
            pad = MP + jnp.bitwise_and(v * 16 + lane, NPAD - 1)
            loc = jnp.where(in_range, iv - lo, pad)
            locbuf[v // (CHUNK // 16), pl.ds((v % (CHUNK // 16)) * 16, 16)] = loc

        for g in range(BPT // CHUNK):
            pltpu.sync_copy(val_hbm.at[pl.ds(jbase + g * CHUNK, CHUNK)],
                            valbuf.at[g % 2])
            pltpu.sync_copy(valbuf.at[g % 2], slab.at[locbuf.at[g]], add=True)

        plsc.subcore_barrier()

        @pl.when(s < NS - 1)
        def _():
            pltpu.sync_copy(slab.at[pl.ds(s * ROWS_A, ROWS_A)],
                            out_hbm.at[pl.ds(off, ROWS_A)])

        @pl.when(s == NS - 1)
        def _():
            pltpu.sync_copy(slab.at[pl.ds(s * ROWS_A, ROWS_B)],
                            out_hbm.at[pl.ds(off, ROWS_B)])


_sc_update = pl.kernel(
    _body,
    out_type=jax.ShapeDtypeStruct((M, D), jnp.float32),
    mesh=plsc.VectorSubcoreMesh(core_axis_name="c", subcore_axis_name="s",
                                num_cores=NC, num_subcores=NS),
    scratch_types=[
        pltpu.VMEM_SHARED((MP + NPAD, D), jnp.float32),
        pltpu.VMEM((2, CHUNK, D), jnp.float32),
        pltpu.VMEM((BPT,), jnp.int32),
        pltpu.VMEM((BPT // CHUNK, CHUNK), jnp.int32),
    ],
    compiler_params=pltpu.CompilerParams(use_tc_tiling_on_sc=False),
)


@jax.jit
def kernel(mem, idx, val):
    return _sc_update(mem, idx.astype(jnp.int32), val)
